# Initial kernel scaffold; baseline (speedup 1.0000x reference)
import jax
import jax.numpy as jnp
from jax.experimental import pallas as pl


def kernel(input, col_w, row_w):
    raise NotImplementedError("write your pallas kernel here")


# speedup vs baseline: 1.4641x; 1.4641x over previous
import jax
import jax.numpy as jnp
from jax.experimental import pallas as pl

_F = 16


def _body(col_t_ref, row_t_ref, out_ref):
    col_t = col_t_ref[...]
    row_t = row_t_ref[...]
    h = row_t.shape[1]
    w = col_t.shape[1]
    out_ref[0, :_F] = jnp.broadcast_to(col_t[:, None, :], (_F, h, w))
    out_ref[0, _F:] = jnp.broadcast_to(row_t[:, :, None], (_F, h, w))


def kernel(input, col_w, row_w):
    b = input.shape[0]
    h, w = input.shape[-2], input.shape[-1]
    f = col_w.shape[-1]
    col_t = col_w.T
    row_t = row_w.T
    return pl.pallas_call(
        _body,
        grid=(b,),
        in_specs=[
            pl.BlockSpec((f, w), lambda i: (0, 0)),
            pl.BlockSpec((f, h), lambda i: (0, 0)),
        ],
        out_specs=pl.BlockSpec((1, 2 * f, h, w), lambda i: (i, 0, 0, 0)),
        out_shape=jax.ShapeDtypeStruct((b, 2 * f, h, w), jnp.float32),
    )(col_t, row_t)


# speedup vs baseline: 1.4807x; 1.0113x over previous
import jax
import jax.numpy as jnp
from jax.experimental import pallas as pl

_F = 16


def _body(col_t_ref, row_t_ref, out_ref):
    col_t = col_t_ref[...]
    row_t = row_t_ref[...]
    h = row_t.shape[1]
    w = col_t.shape[1]
    part = pl.program_id(0) % 2

    @pl.when(part == 0)
    def _():
        out_ref[0] = jnp.broadcast_to(col_t[:, None, :], (_F, h, w))

    @pl.when(part == 1)
    def _():
        out_ref[0] = jnp.broadcast_to(row_t[:, :, None], (_F, h, w))


def kernel(input, col_w, row_w):
    b = input.shape[0]
    h, w = input.shape[-2], input.shape[-1]
    f = col_w.shape[-1]
    col_t = col_w.T
    row_t = row_w.T
    return pl.pallas_call(
        _body,
        grid=(2 * b,),
        in_specs=[
            pl.BlockSpec((f, w), lambda i: (0, 0)),
            pl.BlockSpec((f, h), lambda i: (0, 0)),
        ],
        out_specs=pl.BlockSpec((1, f, h, w), lambda i: (i // 2, i % 2, 0, 0)),
        out_shape=jax.ShapeDtypeStruct((b, 2 * f, h, w), jnp.float32),
    )(col_t, row_t)
